# baseline (speedup 1.0000x reference)
import jax
import jax.numpy as jnp
from jax import lax
from jax.experimental import pallas as pl
from jax.experimental.pallas import tpu as pltpu


def _pick_group(F, H):
    for d in range(1, F + 1):
        if F % d == 0 and d * H >= 256:
            return d
    return F


def _body(a_ref, ash_ref, w1_ref, b1_ref, w2_ref, b2_ref, w3_ref, b3_ref,
          bfrow_ref, rp_ref, base_ref, thr_ref, o_ref,
          h1s_ref, xfix_ref, w1c_ref, w2d_ref, w3d_ref):
    s = pl.program_id(0)
    B, K = a_ref.shape
    G = w1_ref.shape[0]
    H = w1_ref.shape[2]
    NW = G * H
    J = min(128 // H, B // 8)
    B4 = B // J

    w1c_ref[...] = jnp.concatenate([w1_ref[g] for g in range(G)], axis=1)

    @pl.when(s == 0)
    def _zero_diag():
        w2d_ref[...] = jnp.zeros_like(w2d_ref)
        w3d_ref[...] = jnp.zeros_like(w3d_ref)

    for g in range(G):
        w2d_ref[g * H:(g + 1) * H, g * H:(g + 1) * H] = w2_ref[g]
        w3d_ref[g * H:(g + 1) * H, g * H:(g + 1) * H] = w3_ref[g]

    w1c = w1c_ref[...]
    p = jnp.dot(a_ref[...], w1c, preferred_element_type=jnp.float32)
    q = jnp.dot(ash_ref[...], w1c, preferred_element_type=jnp.float32)
    bf = bfrow_ref[0]
    rp = rp_ref[...]
    h1s_ref[...] = jnp.where(rp < bf, p, q)

    for g in range(G):
        base = pl.multiple_of(base_ref[s, g], 8 * J)
        thr = thr_ref[s, g]
        r0 = pl.multiple_of(base // J, 8)
        for j in range(J):
            apj = a_ref[pl.ds(r0 + B4 * j, 8), :]
            ashpj = ash_ref[pl.ds(r0 + B4 * j, 8), :]
            borig = base + J * lax.broadcasted_iota(jnp.int32, (8, K), 0) + j
            n8 = borig * K + lax.broadcasted_iota(jnp.int32, (8, K), 1)
            xfix_ref[8 * (J * g + j):8 * (J * g + j + 1), :] = (
                jnp.where(n8 < thr, apj, ashpj))
    fix = jnp.dot(xfix_ref[...], w1c,
                  preferred_element_type=jnp.float32)
    lane = lax.broadcasted_iota(jnp.int32, (8, NW), 1)
    for g in range(G):
        base = pl.multiple_of(base_ref[s, g], 8 * J)
        r0 = pl.multiple_of(base // J, 8)
        m = (lane >= g * H) & (lane < (g + 1) * H)
        for j in range(J):
            win = h1s_ref[pl.ds(r0 + B4 * j, 8), :]
            h1s_ref[pl.ds(r0 + B4 * j, 8), :] = jnp.where(
                m, fix[8 * (J * g + j):8 * (J * g + j + 1), :], win)

    h = jnp.maximum(h1s_ref[...] + b1_ref[0], 0.0)
    h = jnp.dot(h, w2d_ref[...], preferred_element_type=jnp.float32) + b2_ref[0]
    h = jnp.maximum(h, 0.0)
    h = jnp.dot(h, w3d_ref[...], preferred_element_type=jnp.float32) + b3_ref[0]
    h = jnp.maximum(h, 0.0)

    for g in range(G):
        t = jnp.concatenate(
            [h[B4 * j:B4 * (j + 1), g * H:(g + 1) * H] for j in range(J)],
            axis=1)
        o_ref[g * B4:(g + 1) * B4, :] = t


def kernel(x, W1, b1, W2, b2, W3, b3):
    F, B = x.shape
    K = F - 1
    H = b1.shape[-1]
    G = _pick_group(F, H)
    S = F // G
    NW = G * H
    J = min(128 // H, B // 8)
    B4 = B // J

    xf = x.reshape(-1)
    A = xf[:B * K].reshape(B, K)
    Ash = xf[B:B + B * K].reshape(B, K)
    Ap = A.reshape(B4, J, K).transpose(1, 0, 2).reshape(B, K)
    Ashp = Ash.reshape(B4, J, K).transpose(1, 0, 2).reshape(B, K)
    permvec = (J * (jnp.arange(B, dtype=jnp.int32) % B4)
               + jnp.arange(B, dtype=jnp.int32) // B4)
    rp = jnp.broadcast_to(permvec[:, None], (B, NW))

    b1c = b1.reshape(S, 1, NW)
    b2c = b2.reshape(S, 1, NW)
    b3c = b3.reshape(S, 1, NW)

    f_all = jnp.arange(F, dtype=jnp.int32)
    t_all = f_all * B
    bf_all = t_all // K
    bfrow = jnp.repeat(bf_all.reshape(S, G), H, axis=1).reshape(S, 1, NW)
    W8 = 8 * J
    basearr = jnp.minimum((bf_all // W8) * W8, B - W8).reshape(S, G)
    thrarr = t_all.reshape(S, G)

    out = pl.pallas_call(
        _body,
        grid=(S,),
        in_specs=[
            pl.BlockSpec((B, K), lambda s: (0, 0)),
            pl.BlockSpec((B, K), lambda s: (0, 0)),
            pl.BlockSpec((G, K, H), lambda s: (s, 0, 0)),
            pl.BlockSpec((1, 1, NW), lambda s: (s, 0, 0)),
            pl.BlockSpec((G, H, H), lambda s: (s, 0, 0)),
            pl.BlockSpec((1, 1, NW), lambda s: (s, 0, 0)),
            pl.BlockSpec((G, H, H), lambda s: (s, 0, 0)),
            pl.BlockSpec((1, 1, NW), lambda s: (s, 0, 0)),
            pl.BlockSpec((1, 1, NW), lambda s: (s, 0, 0)),
            pl.BlockSpec((B, NW), lambda s: (0, 0)),
            pl.BlockSpec(memory_space=pltpu.SMEM),
            pl.BlockSpec(memory_space=pltpu.SMEM),
        ],
        out_specs=pl.BlockSpec((G * B4, J * H), lambda s: (s, 0)),
        out_shape=jax.ShapeDtypeStruct((F * B4, J * H), jnp.float32),
        scratch_shapes=[
            pltpu.VMEM((B, NW), jnp.float32),
            pltpu.VMEM((8 * J * G, K), jnp.float32),
            pltpu.VMEM((K, NW), jnp.float32),
            pltpu.VMEM((NW, NW), jnp.float32),
            pltpu.VMEM((NW, NW), jnp.float32),
        ],
        compiler_params=pltpu.CompilerParams(
            dimension_semantics=("arbitrary",),
        ),
        name="feature_loo_mlp_perm",
    )(Ap, Ashp, W1, b1c, W2, b2c, W3, b3c, bfrow, rp, basearr, thrarr)

    return out.reshape(-1)


# speedup vs baseline: 1.0878x; 1.0878x over previous
import jax
import jax.numpy as jnp
from jax import lax
from jax.experimental import pallas as pl
from jax.experimental.pallas import tpu as pltpu


def _pick_group(F, H):
    for d in range(1, F + 1):
        if F % d == 0 and d * H >= 256:
            return d
    return F


def _body(a_ref, ash_ref, w1_ref, b1_ref, w2_ref, b2_ref, w3_ref, b3_ref,
          bfrow_ref, rp_ref, base_ref, thr_ref, o_ref,
          h1s_ref, xfix_ref, w1c_ref, w2d_ref, w3d_ref):
    s = pl.program_id(0)
    B, K = a_ref.shape
    G = w1_ref.shape[0]
    H = w1_ref.shape[2]
    NW = G * H
    J = min(128 // H, B // 8)
    B4 = B // J

    w1c_ref[...] = jnp.concatenate([w1_ref[g] for g in range(G)], axis=1)

    @pl.when(s == 0)
    def _zero_diag():
        w2d_ref[...] = jnp.zeros_like(w2d_ref)
        w3d_ref[...] = jnp.zeros_like(w3d_ref)

    for g in range(G):
        w2d_ref[g * H:(g + 1) * H, g * H:(g + 1) * H] = w2_ref[g]
        w3d_ref[g * H:(g + 1) * H, g * H:(g + 1) * H] = w3_ref[g]

    w1c = w1c_ref[...]
    bf = bfrow_ref[0]
    RC = min(256, B)
    for r in range(0, B, RC):
        pc = jnp.dot(a_ref[r:r + RC, :], w1c,
                     preferred_element_type=jnp.float32)
        qc = jnp.dot(ash_ref[r:r + RC, :], w1c,
                     preferred_element_type=jnp.float32)
        h1s_ref[r:r + RC, :] = jnp.where(rp_ref[r:r + RC, :] < bf, pc, qc)

    for g in range(G):
        base = pl.multiple_of(base_ref[s, g], 8 * J)
        thr = thr_ref[s, g]
        r0 = pl.multiple_of(base // J, 8)
        for j in range(J):
            apj = a_ref[pl.ds(r0 + B4 * j, 8), :]
            ashpj = ash_ref[pl.ds(r0 + B4 * j, 8), :]
            borig = base + J * lax.broadcasted_iota(jnp.int32, (8, K), 0) + j
            n8 = borig * K + lax.broadcasted_iota(jnp.int32, (8, K), 1)
            xfix_ref[8 * (J * g + j):8 * (J * g + j + 1), :] = (
                jnp.where(n8 < thr, apj, ashpj))
    fix = jnp.dot(xfix_ref[...], w1c,
                  preferred_element_type=jnp.float32)
    lane = lax.broadcasted_iota(jnp.int32, (8, NW), 1)
    for g in range(G):
        base = pl.multiple_of(base_ref[s, g], 8 * J)
        r0 = pl.multiple_of(base // J, 8)
        m = (lane >= g * H) & (lane < (g + 1) * H)
        for j in range(J):
            win = h1s_ref[pl.ds(r0 + B4 * j, 8), :]
            h1s_ref[pl.ds(r0 + B4 * j, 8), :] = jnp.where(
                m, fix[8 * (J * g + j):8 * (J * g + j + 1), :], win)

    for j in range(J):
        h = jnp.maximum(h1s_ref[B4 * j:B4 * (j + 1), :] + b1_ref[0], 0.0)
        h = (jnp.dot(h, w2d_ref[...], preferred_element_type=jnp.float32)
             + b2_ref[0])
        h = jnp.maximum(h, 0.0)
        h = (jnp.dot(h, w3d_ref[...], preferred_element_type=jnp.float32)
             + b3_ref[0])
        h = jnp.maximum(h, 0.0)
        for g in range(G):
            o_ref[g * B4:(g + 1) * B4, j * H:(j + 1) * H] = (
                h[:, g * H:(g + 1) * H])


def kernel(x, W1, b1, W2, b2, W3, b3):
    F, B = x.shape
    K = F - 1
    H = b1.shape[-1]
    G = _pick_group(F, H)
    S = F // G
    NW = G * H
    J = min(128 // H, B // 8)
    B4 = B // J

    xf = x.reshape(-1)
    A = xf[:B * K].reshape(B, K)
    Ash = xf[B:B + B * K].reshape(B, K)
    Ap = A.reshape(B4, J, K).transpose(1, 0, 2).reshape(B, K)
    Ashp = Ash.reshape(B4, J, K).transpose(1, 0, 2).reshape(B, K)
    permvec = (J * (jnp.arange(B, dtype=jnp.int32) % B4)
               + jnp.arange(B, dtype=jnp.int32) // B4)
    rp = jnp.broadcast_to(permvec[:, None], (B, NW))

    b1c = b1.reshape(S, 1, NW)
    b2c = b2.reshape(S, 1, NW)
    b3c = b3.reshape(S, 1, NW)

    f_all = jnp.arange(F, dtype=jnp.int32)
    t_all = f_all * B
    bf_all = t_all // K
    bfrow = jnp.repeat(bf_all.reshape(S, G), H, axis=1).reshape(S, 1, NW)
    W8 = 8 * J
    basearr = jnp.minimum((bf_all // W8) * W8, B - W8).reshape(S, G)
    thrarr = t_all.reshape(S, G)

    out = pl.pallas_call(
        _body,
        grid=(S,),
        in_specs=[
            pl.BlockSpec((B, K), lambda s: (0, 0)),
            pl.BlockSpec((B, K), lambda s: (0, 0)),
            pl.BlockSpec((G, K, H), lambda s: (s, 0, 0)),
            pl.BlockSpec((1, 1, NW), lambda s: (s, 0, 0)),
            pl.BlockSpec((G, H, H), lambda s: (s, 0, 0)),
            pl.BlockSpec((1, 1, NW), lambda s: (s, 0, 0)),
            pl.BlockSpec((G, H, H), lambda s: (s, 0, 0)),
            pl.BlockSpec((1, 1, NW), lambda s: (s, 0, 0)),
            pl.BlockSpec((1, 1, NW), lambda s: (s, 0, 0)),
            pl.BlockSpec((B, NW), lambda s: (0, 0)),
            pl.BlockSpec(memory_space=pltpu.SMEM),
            pl.BlockSpec(memory_space=pltpu.SMEM),
        ],
        out_specs=pl.BlockSpec((G * B4, J * H), lambda s: (s, 0)),
        out_shape=jax.ShapeDtypeStruct((F * B4, J * H), jnp.float32),
        scratch_shapes=[
            pltpu.VMEM((B, NW), jnp.float32),
            pltpu.VMEM((8 * J * G, K), jnp.float32),
            pltpu.VMEM((K, NW), jnp.float32),
            pltpu.VMEM((NW, NW), jnp.float32),
            pltpu.VMEM((NW, NW), jnp.float32),
        ],
        compiler_params=pltpu.CompilerParams(
            dimension_semantics=("arbitrary",),
        ),
        name="feature_loo_mlp_perm",
    )(Ap, Ashp, W1, b1c, W2, b2c, W3, b3c, bfrow, rp, basearr, thrarr)

    return out.reshape(-1)
